# baseline (speedup 1.0000x reference)
import functools

import jax
import jax.numpy as jnp
from jax import lax
from jax.experimental import pallas as pl
from jax.experimental.pallas import tpu as pltpu
from jax.experimental.pallas import tpu_sc as plsc

EPSK = 1e-7
INFK = 100000.0
PP = 20480
GP = 56
CH = 2560
NCH = PP // CH
TOPK = 10
NEG = -3.4e38


def _build_kernel(decT_ref, pr_ref, tgt_ref, out_ref, *, G):
    dec = decT_ref[0]
    pr = pr_ref[...]
    tgt = tgt_ref[0]
    gx1 = tgt[:, 0:1]
    gy1 = tgt[:, 1:2]
    gx2 = tgt[:, 2:3]
    gy2 = tgt[:, 3:4]
    gcx = (gx1 + gx2) * 0.5
    gcy = (gy1 + gy2) * 0.5
    ga = jnp.maximum(gx2 - gx1, 0.0) * jnp.maximum(gy2 - gy1, 0.0)
    gi = lax.broadcasted_iota(jnp.int32, (GP, 1), 0)
    gmask = gi < G
    sc = dec[0:1]
    bx1 = dec[1:2]
    by1 = dec[2:3]
    bx2 = dec[3:4]
    by2 = dec[4:5]
    px = pr[0:1] + pr[2:3] * 0.5
    py = pr[1:2] + pr[3:4] * 0.5
    sx = pr[2:3]
    sy = pr[3:4]
    in_gt = (px > gx1) & (py > gy1) & (px < gx2) & (py < gy2)
    in_ct = ((px > gcx - 2.5 * sx) & (py > gcy - 2.5 * sy)
             & (px < gcx + 2.5 * sx) & (py < gcy + 2.5 * sy))
    vi = jnp.where(in_gt | in_ct, 1.0, 0.0)
    valid_c = jnp.max(vi, axis=0, keepdims=True) > 0.0
    iw = jnp.maximum(jnp.minimum(bx2, gx2) - jnp.maximum(bx1, gx1), 0.0)
    ih = jnp.maximum(jnp.minimum(by2, gy2) - jnp.maximum(by1, gy1), 0.0)
    inter = iw * ih
    a1 = jnp.maximum(bx2 - bx1, 0.0) * jnp.maximum(by2 - by1, 0.0)
    union = a1 + ga - inter
    iou = inter / jnp.maximum(union, EPSK)
    icost = -jnp.log(iou + EPSK)
    sq = jnp.sqrt(jnp.clip(sc, EPSK, 1.0))
    cls = -jnp.log(jnp.clip(sq, EPSK, 1.0))
    cost = cls + 3.0 * icost + jnp.where(in_gt & in_ct, 0.0, INFK)
    cost = jnp.where(valid_c, cost, 1e10)
    cost = jnp.where(gmask, cost, 1e30)
    ioum = jnp.where(valid_c, iou, 0.0)
    ioum = jnp.where(gmask, ioum, -1.0)
    out_ref[0, 0] = cost
    out_ref[0, 1] = ioum


def _build(decT, prT, tgt, B):
    return pl.pallas_call(
        functools.partial(_build_kernel, G=50),
        grid=(B, NCH),
        in_specs=[
            pl.BlockSpec((1, 8, CH), lambda n, c: (n, 0, c)),
            pl.BlockSpec((8, CH), lambda n, c: (0, c)),
            pl.BlockSpec((1, GP, 128), lambda n, c: (n, 0, 0)),
        ],
        out_specs=pl.BlockSpec((1, 2, GP, CH), lambda n, c: (n, 0, 0, c)),
        out_shape=jax.ShapeDtypeStruct((B, 2, GP, PP), jnp.float32),
        compiler_params=pltpu.CompilerParams(
            dimension_semantics=("arbitrary", "arbitrary")),
    )(decT, prT, tgt)


def _select_sc(costiou, B, G):
    NTASK = B * 2 * G
    info = plsc.get_sparse_core_info()
    NC = info.num_cores
    NS = info.num_subcores
    NW = NC * NS
    per_w = NTASK // NW
    assert per_w * NW == NTASK
    NGRP = PP // 128

    mesh = plsc.VectorSubcoreMesh(core_axis_name="c", subcore_axis_name="s")
    gdn = lax.GatherDimensionNumbers(
        offset_dims=(), collapsed_slice_dims=(0,), start_index_map=(0,))

    @functools.partial(
        pl.kernel, mesh=mesh,
        out_type=jax.ShapeDtypeStruct((B, 2, GP, 16), jnp.float32),
        compiler_params=pltpu.CompilerParams(needs_layout_passes=False),
        scratch_types=[
            pltpu.VMEM((PP,), jnp.float32),
            pltpu.VMEM((16,), jnp.float32),
        ],
    )
    def sel(src_hbm, out_hbm, col_v, res_v):
        wid = lax.axis_index("s") * NC + lax.axis_index("c")
        idx6 = jnp.full((16, 1), 6, jnp.int32)

        def splat6(v):
            return lax.gather(v, idx6, gdn, (1,),
                              mode=lax.GatherScatterMode.PROMISE_IN_BOUNDS)

        def any_gt(k, thrv):
            return jnp.max(jnp.where(k > thrv, 1.0, 0.0)) > 0.0

        def merge(T, k):
            ks = jnp.sort(k)
            rk = lax.rev(ks, (0,))
            return jnp.sort(jnp.maximum(T, rk))

        def task_body(i, _):
            tau = wid * per_w + i
            b = tau // (2 * G)
            rr = tau % (2 * G)
            m = rr // G
            g = rr % G
            pltpu.sync_copy(src_hbm.at[b, m, g], col_v)
            sgn = jnp.where(m == 0, jnp.float32(-1.0), jnp.float32(1.0))

            def gbody(j, carry):
                T, thrv = carry
                base = pl.multiple_of(j * 128, 128)
                ks = [col_v[pl.ds(base + q * 16, 16)] * sgn
                      for q in range(8)]
                gm = ks[0]
                for q in range(1, 8):
                    gm = jnp.maximum(gm, ks[q])

                def hit(c):
                    T0, thrv0 = c
                    Tn = T0
                    for q in range(8):
                        Tn = lax.cond(any_gt(ks[q], thrv0),
                                      lambda t, kq=ks[q]: merge(t, kq),
                                      lambda t: t, Tn)
                    return Tn, splat6(Tn)

                return lax.cond(any_gt(gm, thrv), hit, lambda c: c, (T, thrv))

            T0 = jnp.full((16,), NEG, jnp.float32)
            thrv0 = jnp.full((16,), NEG, jnp.float32)
            T, _ = lax.fori_loop(0, NGRP, gbody, (T0, thrv0))
            res_v[...] = sgn * lax.rev(T, (0,))
            pltpu.sync_copy(res_v, out_hbm.at[b, m, g])
            return 0

        lax.fori_loop(0, per_w, task_body, 0)

    return sel(costiou)


def _fin_kernel(decT_ref, predT_ref, tgt_ref, ci_ref, ii_ref, top_ref,
                out_ref, *, G, P_REAL, NIMG):
    n = pl.program_id(0)
    dec = decT_ref[0]
    tgt = tgt_ref[0]
    gx1 = tgt[:, 0:1]
    gy1 = tgt[:, 1:2]
    gx2 = tgt[:, 2:3]
    gy2 = tgt[:, 3:4]
    gi = lax.broadcasted_iota(jnp.int32, (GP, 1), 0)
    liF = lax.broadcasted_iota(jnp.int32, (1, PP), 1)
    top = top_ref[0]
    ctop = top[0]
    itop = top[1]
    ksum = jnp.zeros((GP, 1), jnp.float32)
    for t in range(TOPK):
        ksum = ksum + jnp.maximum(itop[:, t:t + 1], 0.0)
    dyn_ks = jnp.maximum(ksum.astype(jnp.int32), 1)
    thresh = jnp.zeros((GP, 1), jnp.float32)
    for t in range(TOPK):
        thresh = thresh + jnp.where(dyn_ks == t + 1, ctop[:, t:t + 1], 0.0)

    f_sum = jnp.float32(0.0)
    e_sum = jnp.float32(0.0)
    npos = jnp.float32(0.0)
    for c in range(NCH):
        s = c * CH
        li = liF[:, s:s + CH]
        C = ci_ref[0, 0, :, s:s + CH]
        I = ii_ref[0, 0, :, s:s + CH]
        M = jnp.where((C <= thresh) & (C < 1e10), 1.0, 0.0)
        mc = jnp.sum(M, axis=0, keepdims=True)
        cmin = jnp.min(C, axis=0, keepdims=True)
        gstar = jnp.min(jnp.where(C == cmin, gi, 1 << 30),
                        axis=0, keepdims=True)
        multi = mc > 1.5
        onehot = jnp.where(gi == gstar, 1.0, 0.0)
        Mf = jnp.where(multi, onehot, M)
        fg = mc > 0.5
        fgf = jnp.where(fg, 1.0, 0.0)
        conf = jnp.sum(Mf * I, axis=0, keepdims=True)
        conf = jnp.where(fg, conf, 0.0)
        tx1 = jnp.sum(Mf * gx1, axis=0, keepdims=True)
        ty1 = jnp.sum(Mf * gy1, axis=0, keepdims=True)
        tx2 = jnp.sum(Mf * gx2, axis=0, keepdims=True)
        ty2 = jnp.sum(Mf * gy2, axis=0, keepdims=True)
        l = predT_ref[0][0:1, s:s + CH]
        real = li < P_REAL
        e = jnp.exp(-jnp.abs(l))
        ce = jnp.maximum(l, 0.0) - l * conf + jnp.log(1.0 + e)
        p = jnp.where(l >= 0.0, 1.0 / (1.0 + e), e / (1.0 + e))
        p_t = p * conf + (1.0 - p) * (1.0 - conf)
        a_t = 0.25 * conf + 0.75 * (1.0 - conf)
        om = 1.0 - p_t
        fterm = a_t * om * om * ce
        f_sum = f_sum + jnp.sum(jnp.where(real, fterm, 0.0))
        bx1 = dec[1:2, s:s + CH]
        by1 = dec[2:3, s:s + CH]
        bx2 = dec[3:4, s:s + CH]
        by2 = dec[4:5, s:s + CH]
        iw = jnp.maximum(jnp.minimum(bx2, tx2) - jnp.maximum(bx1, tx1), 0.0)
        ih = jnp.maximum(jnp.minimum(by2, ty2) - jnp.maximum(by1, ty1), 0.0)
        inter = iw * ih
        ap_ = jnp.maximum(bx2 - bx1, 0.0) * jnp.maximum(by2 - by1, 0.0)
        at_ = jnp.maximum(tx2 - tx1, 0.0) * jnp.maximum(ty2 - ty1, 0.0)
        union = ap_ + at_ - inter
        iou_e = inter / (union + EPSK)
        cw = jnp.maximum(bx2, tx2) - jnp.minimum(bx1, tx1)
        chh = jnp.maximum(by2, ty2) - jnp.minimum(by1, ty1)
        c2 = cw * cw + chh * chh + EPSK
        dx = (bx1 + bx2) * 0.5 - (tx1 + tx2) * 0.5
        dy = (by1 + by2) * 0.5 - (ty1 + ty2) * 0.5
        rho2 = dx * dx + dy * dy
        dw = (bx2 - bx1) - (tx2 - tx1)
        dh = (by2 - by1) - (ty2 - ty1)
        elem = (1.0 - iou_e + rho2 / c2
                + dw * dw / (cw * cw + EPSK) + dh * dh / (chh * chh + EPSK))
        e_sum = e_sum + jnp.sum(elem * fgf)
        npos = npos + jnp.sum(fgf)

    cl = f_sum / P_REAL
    bl = jnp.where(npos > 0.0, e_sum / jnp.maximum(npos, 1.0), 0.0)
    loss_n = (cl + 5.0 * bl) / NIMG

    @pl.when(n == 0)
    def _():
        out_ref[...] = jnp.zeros_like(out_ref)

    out_ref[...] = out_ref[...] + loss_n * jnp.ones((8, 128), jnp.float32)


def kernel(predictions, priors, decoded_bboxes, targets):
    B, P, _ = predictions.shape
    G = targets.shape[1]
    decT = jnp.transpose(decoded_bboxes, (0, 2, 1))
    decT = jnp.pad(decT, ((0, 0), (0, 3), (0, PP - P)))
    prT = jnp.transpose(priors, (1, 0))
    prT = jnp.pad(prT, ((0, 4), (0, PP - P)))
    prT = prT.at[0:2, P:].set(-1e7)
    prT = prT.at[2:4, P:].set(1.0)
    predT = jnp.transpose(predictions, (0, 2, 1))
    predT = jnp.pad(predT, ((0, 0), (0, 7), (0, PP - P)))
    tgt = jnp.pad(targets, ((0, 0), (0, GP - G), (0, 128 - 4)),
                  constant_values=-1e6)
    costiou = _build(decT, prT, tgt, B)
    top = _select_sc(costiou, B, G)
    out = pl.pallas_call(
        functools.partial(_fin_kernel, G=G, P_REAL=P, NIMG=B),
        grid=(B,),
        in_specs=[
            pl.BlockSpec((1, 8, PP), lambda n: (n, 0, 0)),
            pl.BlockSpec((1, 8, PP), lambda n: (n, 0, 0)),
            pl.BlockSpec((1, GP, 128), lambda n: (n, 0, 0)),
            pl.BlockSpec((1, 1, GP, PP), lambda n: (n, 0, 0, 0)),
            pl.BlockSpec((1, 1, GP, PP), lambda n: (n, 1, 0, 0)),
            pl.BlockSpec((1, 2, GP, 16), lambda n: (n, 0, 0, 0)),
        ],
        out_specs=pl.BlockSpec((8, 128), lambda n: (0, 0)),
        out_shape=jax.ShapeDtypeStruct((8, 128), jnp.float32),
        compiler_params=pltpu.CompilerParams(
            dimension_semantics=("arbitrary",)),
    )(decT, predT, tgt, costiou, costiou, top)
    return out[0, 0]


# speedup vs baseline: 1.8192x; 1.8192x over previous
import functools

import jax
import jax.numpy as jnp
from jax import lax
from jax.experimental import pallas as pl
from jax.experimental.pallas import tpu as pltpu

EPSK = 1e-7
INFK = 100000.0
PP = 20480
GP = 56
CH = 2560
NCH = PP // CH
TOPK = 10
BIGI = 1 << 30


def _loss_kernel(decT_ref, pr_ref, predT_ref, tgt_ref, out_ref,
                 cost_ref, iou_ref, *, G, P_REAL, NIMG):
    n = pl.program_id(0)
    dec = decT_ref[0]
    pr = pr_ref[...]
    tgt = tgt_ref[0]
    gx1 = tgt[:, 0:1]
    gy1 = tgt[:, 1:2]
    gx2 = tgt[:, 2:3]
    gy2 = tgt[:, 3:4]
    gcx = (gx1 + gx2) * 0.5
    gcy = (gy1 + gy2) * 0.5
    ga = jnp.maximum(gx2 - gx1, 0.0) * jnp.maximum(gy2 - gy1, 0.0)
    gi = lax.broadcasted_iota(jnp.int32, (GP, 1), 0)
    gmask = gi < G
    liF = lax.broadcasted_iota(jnp.int32, (1, PP), 1)

    cmin0 = None
    imax0 = None
    for c in range(NCH):
        s = c * CH
        sc = dec[0:1, s:s + CH]
        bx1 = dec[1:2, s:s + CH]
        by1 = dec[2:3, s:s + CH]
        bx2 = dec[3:4, s:s + CH]
        by2 = dec[4:5, s:s + CH]
        sx = pr[2:3, s:s + CH]
        sy = pr[3:4, s:s + CH]
        px = pr[0:1, s:s + CH] + sx * 0.5
        py = pr[1:2, s:s + CH] + sy * 0.5
        in_gt = (px > gx1) & (py > gy1) & (px < gx2) & (py < gy2)
        in_ct = ((px > gcx - 2.5 * sx) & (py > gcy - 2.5 * sy)
                 & (px < gcx + 2.5 * sx) & (py < gcy + 2.5 * sy))
        vi = jnp.where(in_gt | in_ct, 1.0, 0.0)
        valid_c = jnp.max(vi, axis=0, keepdims=True) > 0.0
        iw = jnp.maximum(jnp.minimum(bx2, gx2) - jnp.maximum(bx1, gx1), 0.0)
        ih = jnp.maximum(jnp.minimum(by2, gy2) - jnp.maximum(by1, gy1), 0.0)
        inter = iw * ih
        a1 = jnp.maximum(bx2 - bx1, 0.0) * jnp.maximum(by2 - by1, 0.0)
        union = a1 + ga - inter
        iou = inter / jnp.maximum(union, EPSK)
        icost = -jnp.log(iou + EPSK)
        sq = jnp.sqrt(jnp.clip(sc, EPSK, 1.0))
        cls = -jnp.log(jnp.clip(sq, EPSK, 1.0))
        cost = cls + 3.0 * icost + jnp.where(in_gt & in_ct, 0.0, INFK)
        cost = jnp.where(valid_c, cost, 1e10)
        cost = jnp.where(gmask, cost, 1e30)
        ioum = jnp.where(valid_c, iou, 0.0)
        ioum = jnp.where(gmask, ioum, -1.0)
        cost_ref[:, s:s + CH] = cost
        iou_ref[:, s:s + CH] = ioum
        ccm = jnp.min(cost, axis=1, keepdims=True)
        cim = jnp.max(ioum, axis=1, keepdims=True)
        cmin0 = ccm if cmin0 is None else jnp.minimum(cmin0, ccm)
        imax0 = cim if imax0 is None else jnp.maximum(imax0, cim)

    ksum = jnp.maximum(imax0, 0.0)
    prev = imax0
    for _t in range(1, TOPK):
        W = iou_ref[...]
        v = jnp.max(jnp.where(W < prev, W, -3.0), axis=1, keepdims=True)
        ksum = ksum + jnp.maximum(v, 0.0)
        prev = v
    dyn_ks = jnp.maximum(ksum.astype(jnp.int32), 1)

    cvals = [cmin0]
    prev = cmin0
    for _t in range(1, TOPK):
        W = cost_ref[...]
        v = jnp.min(jnp.where(W > prev, W, 1e35), axis=1, keepdims=True)
        cvals.append(v)
        prev = v
    thresh = jnp.zeros((GP, 1), jnp.float32)
    for t in range(TOPK):
        thresh = thresh + jnp.where(dyn_ks == t + 1, cvals[t], 0.0)

    f_sum = jnp.float32(0.0)
    e_sum = jnp.float32(0.0)
    npos = jnp.float32(0.0)
    for c in range(NCH):
        s = c * CH
        li = liF[:, s:s + CH]
        C = cost_ref[:, s:s + CH]
        I = iou_ref[:, s:s + CH]
        M = jnp.where((C <= thresh) & (C < 1e10), 1.0, 0.0)
        mc = jnp.sum(M, axis=0, keepdims=True)
        cmin = jnp.min(C, axis=0, keepdims=True)
        gstar = jnp.min(jnp.where(C == cmin, gi, BIGI), axis=0, keepdims=True)
        multi = mc > 1.5
        onehot = jnp.where(gi == gstar, 1.0, 0.0)
        Mf = jnp.where(multi, onehot, M)
        fg = mc > 0.5
        fgf = jnp.where(fg, 1.0, 0.0)
        conf = jnp.sum(Mf * I, axis=0, keepdims=True)
        conf = jnp.where(fg, conf, 0.0)
        tx1 = jnp.sum(Mf * gx1, axis=0, keepdims=True)
        ty1 = jnp.sum(Mf * gy1, axis=0, keepdims=True)
        tx2 = jnp.sum(Mf * gx2, axis=0, keepdims=True)
        ty2 = jnp.sum(Mf * gy2, axis=0, keepdims=True)
        l = predT_ref[0][0:1, s:s + CH]
        real = li < P_REAL
        e = jnp.exp(-jnp.abs(l))
        ce = jnp.maximum(l, 0.0) - l * conf + jnp.log(1.0 + e)
        p = jnp.where(l >= 0.0, 1.0 / (1.0 + e), e / (1.0 + e))
        p_t = p * conf + (1.0 - p) * (1.0 - conf)
        a_t = 0.25 * conf + 0.75 * (1.0 - conf)
        om = 1.0 - p_t
        fterm = a_t * om * om * ce
        f_sum = f_sum + jnp.sum(jnp.where(real, fterm, 0.0))
        bx1 = dec[1:2, s:s + CH]
        by1 = dec[2:3, s:s + CH]
        bx2 = dec[3:4, s:s + CH]
        by2 = dec[4:5, s:s + CH]
        iw = jnp.maximum(jnp.minimum(bx2, tx2) - jnp.maximum(bx1, tx1), 0.0)
        ih = jnp.maximum(jnp.minimum(by2, ty2) - jnp.maximum(by1, ty1), 0.0)
        inter = iw * ih
        ap_ = jnp.maximum(bx2 - bx1, 0.0) * jnp.maximum(by2 - by1, 0.0)
        at_ = jnp.maximum(tx2 - tx1, 0.0) * jnp.maximum(ty2 - ty1, 0.0)
        union = ap_ + at_ - inter
        iou_e = inter / (union + EPSK)
        cw = jnp.maximum(bx2, tx2) - jnp.minimum(bx1, tx1)
        chh = jnp.maximum(by2, ty2) - jnp.minimum(by1, ty1)
        c2 = cw * cw + chh * chh + EPSK
        dx = (bx1 + bx2) * 0.5 - (tx1 + tx2) * 0.5
        dy = (by1 + by2) * 0.5 - (ty1 + ty2) * 0.5
        rho2 = dx * dx + dy * dy
        dw = (bx2 - bx1) - (tx2 - tx1)
        dh = (by2 - by1) - (ty2 - ty1)
        elem = (1.0 - iou_e + rho2 / c2
                + dw * dw / (cw * cw + EPSK) + dh * dh / (chh * chh + EPSK))
        e_sum = e_sum + jnp.sum(elem * fgf)
        npos = npos + jnp.sum(fgf)

    cl = f_sum / P_REAL
    bl = jnp.where(npos > 0.0, e_sum / jnp.maximum(npos, 1.0), 0.0)
    loss_n = (cl + 5.0 * bl) / NIMG

    @pl.when(n == 0)
    def _():
        out_ref[...] = jnp.zeros_like(out_ref)

    out_ref[...] = out_ref[...] + loss_n * jnp.ones((8, 128), jnp.float32)


def kernel(predictions, priors, decoded_bboxes, targets):
    B, P, _ = predictions.shape
    G = targets.shape[1]
    decT = jnp.transpose(decoded_bboxes, (0, 2, 1))
    decT = jnp.pad(decT, ((0, 0), (0, 3), (0, PP - P)))
    prT = jnp.transpose(priors, (1, 0))
    prT = jnp.pad(prT, ((0, 4), (0, PP - P)))
    prT = prT.at[0:2, P:].set(-1e7)
    prT = prT.at[2:4, P:].set(1.0)
    predT = jnp.transpose(predictions, (0, 2, 1))
    predT = jnp.pad(predT, ((0, 0), (0, 7), (0, PP - P)))
    tgt = jnp.pad(targets, ((0, 0), (0, GP - G), (0, 128 - 4)),
                  constant_values=-1e6)
    out = pl.pallas_call(
        functools.partial(_loss_kernel, G=G, P_REAL=P, NIMG=B),
        grid=(B,),
        in_specs=[
            pl.BlockSpec((1, 8, PP), lambda n: (n, 0, 0)),
            pl.BlockSpec((8, PP), lambda n: (0, 0)),
            pl.BlockSpec((1, 8, PP), lambda n: (n, 0, 0)),
            pl.BlockSpec((1, GP, 128), lambda n: (n, 0, 0)),
        ],
        out_specs=pl.BlockSpec((8, 128), lambda n: (0, 0)),
        out_shape=jax.ShapeDtypeStruct((8, 128), jnp.float32),
        scratch_shapes=[
            pltpu.VMEM((GP, PP), jnp.float32),
            pltpu.VMEM((GP, PP), jnp.float32),
        ],
        compiler_params=pltpu.CompilerParams(
            dimension_semantics=("arbitrary",)),
    )(decT, prT, predT, tgt)
    return out[0, 0]
